# baseline (speedup 1.0000x reference)
import functools

import jax
import jax.numpy as jnp
from jax.experimental import pallas as pl
from jax.experimental.pallas import tpu as pltpu


_RC = 8


def _knn_interp_kernel(xyz1t_ref, xyz2_ref, p2_ref, out_ref, wmat_ref):
    k = xyz2_ref[0]
    p2 = p2_ref[0]
    N = xyz1t_ref.shape[1]
    S = k.shape[1]
    kb = k.astype(jnp.bfloat16).astype(jnp.float32)
    n2 = k[0:1, :] * k[0:1, :]
    n2 = n2 + k[1:2, :] * k[1:2, :]
    n2 = n2 + k[2:3, :] * k[2:3, :]
    lane = jax.lax.broadcasted_iota(jnp.int32, (_RC, S), 1)

    def body(i, carry):
        q = xyz1t_ref[0, pl.ds(i * _RC, _RC), :]
        qb = q.astype(jnp.bfloat16).astype(jnp.float32)
        qk = qb[:, 0:1] * kb[0:1, :]
        qk = qk + qb[:, 1:2] * kb[1:2, :]
        qk = qk + qb[:, 2:3] * kb[2:3, :]
        n1 = q[:, 0:1] * q[:, 0:1]
        n1 = n1 + q[:, 1:2] * q[:, 1:2]
        n1 = n1 + q[:, 2:3] * q[:, 2:3]
        d = -2.0 * qk
        d = d + n1
        d = d + n2
        mvs = []
        wacc = jnp.zeros((_RC, S), dtype=jnp.float32)
        masks = []
        for _ in range(3):
            mv = jnp.min(d, axis=1, keepdims=True)
            t = jnp.where(d == mv, lane, S)
            idx = jnp.min(t, axis=1, keepdims=True)
            eqm = t == idx
            mvs.append(mv)
            masks.append(eqm)
            d = jnp.where(eqm, jnp.inf, d)
        r = [1.0 / (mv + 1e-8) for mv in mvs]
        norm = r[0] + r[1] + r[2]
        for kk in range(3):
            wacc = jnp.where(masks[kk], r[kk] / norm, wacc)
        wmat_ref[pl.ds(i * _RC, _RC), :] = wacc
        return carry

    jax.lax.fori_loop(0, N // _RC, body, 0)
    interp = jax.lax.dot_general(
        wmat_ref[...], p2, (((1,), (1,)), ((), ())),
        preferred_element_type=jnp.float32,
        precision=jax.lax.Precision.HIGHEST)
    out_ref[:, 0, 0, :] = interp


def _bn_relu(x1, b_ref, g_ref, be_ref, out_ref):
    x1 = x1 + b_ref[...]
    bl = x1.shape[1]
    m = jnp.sum(x1, axis=1, keepdims=True) / bl
    xc = x1 - m
    v = jnp.sum(xc * xc, axis=1, keepdims=True) / bl
    xh = xc * jax.lax.rsqrt(v + 1e-5)
    y = g_ref[...] * xh + be_ref[...]
    out_ref[...] = jnp.maximum(y, 0.0)


def _mm_bn_relu_kernel(w_ref, x_ref, b_ref, g_ref, be_ref, out_ref):
    x1 = jax.lax.dot_general(
        w_ref[...], x_ref[...], (((1,), (0,)), ((), ())),
        preferred_element_type=jnp.float32)
    _bn_relu(x1, b_ref, g_ref, be_ref, out_ref)


def _mm2_bn_relu_kernel(wa_ref, xa_ref, wb_ref, xb_ref, b_ref, g_ref, be_ref,
                        out_ref):
    x1 = jax.lax.dot_general(
        wa_ref[...], xa_ref[...], (((1,), (0,)), ((), ())),
        preferred_element_type=jnp.float32)
    x1 = x1 + jax.lax.dot_general(
        wb_ref[...], xb_ref[...], (((1,), (0,)), ((), ())),
        preferred_element_type=jnp.float32)
    _bn_relu(x1, b_ref, g_ref, be_ref, out_ref)


def kernel(xyz1, xyz2, points1, points2, W1, b1, g1, be1, W2, b2, g2, be2):
    B, _, N = xyz1.shape
    S = xyz2.shape[2]
    D = points2.shape[1]
    c1 = W1.shape[0]
    c2 = W2.shape[0]
    BL = B * D

    xyz1t = jnp.transpose(xyz1, (0, 2, 1))

    interp_t = pl.pallas_call(
        _knn_interp_kernel,
        grid=(B,),
        in_specs=[
            pl.BlockSpec((1, N, 3), lambda b: (b, 0, 0)),
            pl.BlockSpec((1, 3, S), lambda b: (b, 0, 0)),
            pl.BlockSpec((1, D, S), lambda b: (b, 0, 0)),
        ],
        out_specs=pl.BlockSpec((N, 1, 1, D), lambda b: (0, b, 0, 0)),
        out_shape=jax.ShapeDtypeStruct((N, B, 1, D), jnp.float32),
        scratch_shapes=[pltpu.VMEM((N, S), jnp.float32)],
    )(xyz1t, xyz2, points2)

    p1t = jnp.transpose(points1, (2, 0, 1)).reshape(N, BL)
    interp2d = interp_t.reshape(N, BL)

    def mm_stage(Ws, b, g, be, xs, rows, row_chunk):
        nblk = rows // row_chunk
        in_specs = []
        args = []
        for W, x in zip(Ws, xs):
            cdim = W.shape[1]
            in_specs.append(pl.BlockSpec((row_chunk, cdim), lambda r: (r, 0)))
            in_specs.append(pl.BlockSpec((cdim, BL), lambda r: (0, 0)))
            args.extend([W, x])
        in_specs.extend([pl.BlockSpec((row_chunk, 1), lambda r: (r, 0))] * 3)
        args.extend([b.reshape(rows, 1), g.reshape(rows, 1), be.reshape(rows, 1)])
        kern = _mm2_bn_relu_kernel if len(Ws) == 2 else _mm_bn_relu_kernel
        return pl.pallas_call(
            kern,
            grid=(nblk,),
            in_specs=in_specs,
            out_specs=pl.BlockSpec((row_chunk, BL), lambda r: (r, 0)),
            out_shape=jax.ShapeDtypeStruct((rows, BL), jnp.float32),
        )(*args)

    y1 = mm_stage((W1[:, :N], W1[:, N:]), b1, g1, be1,
                  (p1t, interp2d), c1, 256)
    y2 = mm_stage((W2,), b2, g2, be2, (y1,), c2, 256)

    return jnp.transpose(y2.reshape(c2, B, D), (1, 0, 2))


# speedup vs baseline: 7.9509x; 7.9509x over previous
import functools

import jax
import jax.numpy as jnp
from jax.experimental import pallas as pl
from jax.experimental.pallas import tpu as pltpu


def _knn_interp_kernel(xyz1t_ref, xyz2_ref, p2_ref, out_ref):
    q = xyz1t_ref[0]
    k = xyz2_ref[0]
    p2 = p2_ref[0]
    N = q.shape[0]
    S = k.shape[1]
    qb = q.astype(jnp.bfloat16).astype(jnp.float32)
    kb = k.astype(jnp.bfloat16).astype(jnp.float32)
    qk = qb[:, 0:1] * kb[0:1, :]
    qk = qk + qb[:, 1:2] * kb[1:2, :]
    qk = qk + qb[:, 2:3] * kb[2:3, :]
    n1 = q[:, 0:1] * q[:, 0:1]
    n1 = n1 + q[:, 1:2] * q[:, 1:2]
    n1 = n1 + q[:, 2:3] * q[:, 2:3]
    n2 = k[0:1, :] * k[0:1, :]
    n2 = n2 + k[1:2, :] * k[1:2, :]
    n2 = n2 + k[2:3, :] * k[2:3, :]
    d = -2.0 * qk
    d = d + n1
    d = d + n2
    lane = jax.lax.broadcasted_iota(jnp.int32, (N, S), 1)
    mvs = []
    masks = []
    for _ in range(3):
        mv = jnp.min(d, axis=1, keepdims=True)
        t = jnp.where(d == mv, lane, S)
        idx = jnp.min(t, axis=1, keepdims=True)
        eqm = t == idx
        mvs.append(mv)
        masks.append(eqm)
        d = jnp.where(eqm, jnp.inf, d)
    r = [1.0 / (mv + 1e-8) for mv in mvs]
    norm = r[0] + r[1] + r[2]
    wacc = jnp.zeros((N, S), dtype=jnp.float32)
    for kk in range(3):
        wacc = jnp.where(masks[kk], r[kk] / norm, wacc)
    interp = jax.lax.dot_general(
        wacc, p2, (((1,), (1,)), ((), ())),
        preferred_element_type=jnp.float32,
        precision=jax.lax.Precision.HIGHEST)
    out_ref[:, 0, 0, :] = interp


def _bn_relu(x1, b_ref, g_ref, be_ref, out_ref):
    x1 = x1 + b_ref[...]
    bl = x1.shape[1]
    m = jnp.sum(x1, axis=1, keepdims=True) / bl
    xc = x1 - m
    v = jnp.sum(xc * xc, axis=1, keepdims=True) / bl
    xh = xc * jax.lax.rsqrt(v + 1e-5)
    y = g_ref[...] * xh + be_ref[...]
    out_ref[...] = jnp.maximum(y, 0.0)


def _mm_bn_relu_kernel(w_ref, x_ref, b_ref, g_ref, be_ref, out_ref):
    x1 = jax.lax.dot_general(
        w_ref[...], x_ref[...], (((1,), (0,)), ((), ())),
        preferred_element_type=jnp.float32)
    _bn_relu(x1, b_ref, g_ref, be_ref, out_ref)


def _mm2_bn_relu_kernel(wa_ref, xa_ref, wb_ref, xb_ref, b_ref, g_ref, be_ref,
                        out_ref):
    x1 = jax.lax.dot_general(
        wa_ref[...], xa_ref[...], (((1,), (0,)), ((), ())),
        preferred_element_type=jnp.float32)
    x1 = x1 + jax.lax.dot_general(
        wb_ref[...], xb_ref[...], (((1,), (0,)), ((), ())),
        preferred_element_type=jnp.float32)
    _bn_relu(x1, b_ref, g_ref, be_ref, out_ref)


def kernel(xyz1, xyz2, points1, points2, W1, b1, g1, be1, W2, b2, g2, be2):
    B, _, N = xyz1.shape
    S = xyz2.shape[2]
    D = points2.shape[1]
    c1 = W1.shape[0]
    c2 = W2.shape[0]
    BL = B * D

    xyz1t = jnp.transpose(xyz1, (0, 2, 1))

    interp_t = pl.pallas_call(
        _knn_interp_kernel,
        grid=(B,),
        in_specs=[
            pl.BlockSpec((1, N, 3), lambda b: (b, 0, 0)),
            pl.BlockSpec((1, 3, S), lambda b: (b, 0, 0)),
            pl.BlockSpec((1, D, S), lambda b: (b, 0, 0)),
        ],
        out_specs=pl.BlockSpec((N, 1, 1, D), lambda b: (0, b, 0, 0)),
        out_shape=jax.ShapeDtypeStruct((N, B, 1, D), jnp.float32),
    )(xyz1t, xyz2, points2)

    p1t = jnp.transpose(points1, (2, 0, 1)).reshape(N, BL)
    interp2d = interp_t.reshape(N, BL)

    def mm_stage(Ws, b, g, be, xs, rows, row_chunk):
        nblk = rows // row_chunk
        in_specs = []
        args = []
        for W, x in zip(Ws, xs):
            cdim = W.shape[1]
            in_specs.append(pl.BlockSpec((row_chunk, cdim), lambda r: (r, 0)))
            in_specs.append(pl.BlockSpec((cdim, BL), lambda r: (0, 0)))
            args.extend([W, x])
        in_specs.extend([pl.BlockSpec((row_chunk, 1), lambda r: (r, 0))] * 3)
        args.extend([b.reshape(rows, 1), g.reshape(rows, 1), be.reshape(rows, 1)])
        kern = _mm2_bn_relu_kernel if len(Ws) == 2 else _mm_bn_relu_kernel
        return pl.pallas_call(
            kern,
            grid=(nblk,),
            in_specs=in_specs,
            out_specs=pl.BlockSpec((row_chunk, BL), lambda r: (r, 0)),
            out_shape=jax.ShapeDtypeStruct((rows, BL), jnp.float32),
        )(*args)

    y1 = mm_stage((W1[:, :N], W1[:, N:]), b1, g1, be1,
                  (p1t, interp2d), c1, 256)
    y2 = mm_stage((W2,), b2, g2, be2, (y1,), c2, 256)

    return jnp.transpose(y2.reshape(c2, B, D), (1, 0, 2))
